# baseline (speedup 1.0000x reference)
import functools

import jax
import jax.numpy as jnp
from jax import lax
from jax.experimental import pallas as pl
from jax.experimental.pallas import tpu as pltpu
from jax.experimental.pallas import tpu_sc as plsc

_BS = 4
_P = 4096
_K = 32
_R2 = 0.2 * 0.2
_L = 16
_NW = 32
_CPW = _BS * _P // _NW
_WPB = _P // _CPW
_NV = _P // _L
_CHUNK = 16
_SUB = 16
_NCH = _NV // _CHUNK
_SCRATCH = 4 * _CHUNK * _L + _L


def _radius_body(xyz_hbm, nbr_hbm, fil_hbm, x_ref, y_ref, z_ref,
                 sc_ref, nbr_v, fil_v, cnt_ref):
    wid = lax.axis_index("s") * 2 + lax.axis_index("c")
    b = wid // _WPB
    c0 = (wid % _WPB) * _CPW

    pltpu.sync_copy(xyz_hbm.at[pl.ds((b * 3 + 0) * _P, _P)],
                    x_ref.at[pl.ds(0, _P)])
    pltpu.sync_copy(xyz_hbm.at[pl.ds((b * 3 + 1) * _P, _P)],
                    y_ref.at[pl.ds(0, _P)])
    pltpu.sync_copy(xyz_hbm.at[pl.ds((b * 3 + 2) * _P, _P)],
                    z_ref.at[pl.ds(0, _P)])

    lanes = lax.iota(jnp.int32, 16)

    def per_center(i, _):
        c = c0 + i
        cxv = jnp.full((16,), x_ref[pl.ds(c, _L)][0], jnp.float32)
        cyv = jnp.full((16,), y_ref[pl.ds(c, _L)][0], jnp.float32)
        czv = jnp.full((16,), z_ref[pl.ds(c, _L)][0], jnp.float32)

        def scan_chunk(base, cnt):
            ms = []
            for u in range(_CHUNK):
                off = base + u * _L
                px = x_ref[pl.ds(off, _L)]
                py = y_ref[pl.ds(off, _L)]
                pz = z_ref[pl.ds(off, _L)]
                dx = px - cxv
                dy = py - cyv
                dz = pz - czv
                ms.append(dx * dx + dy * dy + dz * dz < _R2)
            for u in range(_CHUNK):
                off = base + u * _L
                plsc.store_compressed(sc_ref.at[pl.ds(cnt, _L)],
                                      off + lanes, mask=ms[u])
                cnt = cnt + plsc.all_reduce_population_count(ms[u])[0]
            return cnt

        cnt = scan_chunk(0, 0)
        cnt = scan_chunk(_CHUNK * _L, cnt)
        cnt = scan_chunk(2 * _CHUNK * _L, cnt)
        cnt = scan_chunk(3 * _CHUNK * _L, cnt)
        cnt_ref[0] = cnt

        def chunk_step(ch, _c):
            @pl.when(cnt_ref[0] < _K)
            def _do_chunk():
                cnt_ref[0] = scan_chunk(ch * (_CHUNK * _L), cnt_ref[0])

            return 0

        lax.fori_loop(4, 6, chunk_step, 0)

        @pl.when(cnt_ref[0] < _K)
        def _tail():
            lax.fori_loop(6, _NCH, chunk_step, 0)
        cnt = jnp.full((16,), cnt_ref[0], jnp.int32)

        v0 = sc_ref[pl.ds(0, _L)]
        v1 = sc_ref[pl.ds(_L, _L)]
        first = jnp.full((16,), v0[0], jnp.int32)
        f0 = lax.shift_right_logical(lanes - cnt, 31)
        f1 = lax.shift_right_logical((lanes + _L) - cnt, 31)
        o = i * _K
        nbr_v[pl.ds(o, _L)] = v0 * f0 + first * (1 - f0)
        nbr_v[pl.ds(o + _L, _L)] = v1 * f1 + first * (1 - f1)
        fil_v[pl.ds(o, _L)] = f0
        fil_v[pl.ds(o + _L, _L)] = f1
        return 0

    lax.fori_loop(0, _CPW, per_center, 0)

    pltpu.sync_copy(nbr_v, nbr_hbm.at[pl.ds(wid * _CPW * _K, _CPW * _K)])
    pltpu.sync_copy(fil_v, fil_hbm.at[pl.ds(wid * _CPW * _K, _CPW * _K)])


_radius_sc = functools.partial(
    pl.kernel,
    mesh=plsc.VectorSubcoreMesh(core_axis_name="c", subcore_axis_name="s"),
    out_type=[
        jax.ShapeDtypeStruct((_BS * _P * _K,), jnp.int32),
        jax.ShapeDtypeStruct((_BS * _P * _K,), jnp.int32),
    ],
    scratch_types=[
        pltpu.VMEM((_P + _L,), jnp.float32),
        pltpu.VMEM((_P + _L,), jnp.float32),
        pltpu.VMEM((_P + _L,), jnp.float32),
        pltpu.VMEM((_SCRATCH,), jnp.int32),
        pltpu.VMEM((_CPW * _K,), jnp.int32),
        pltpu.VMEM((_CPW * _K,), jnp.int32),
        pltpu.SMEM((1,), jnp.int32),
    ],
    compiler_params=pltpu.CompilerParams(needs_layout_passes=False),
)(_radius_body)


def kernel(xyz):
    bs, p = xyz.shape[:2]
    xyz_t = jnp.transpose(xyz, (0, 2, 1)).reshape(-1)
    nbr_f, fil_f = _radius_sc(xyz_t)
    nbr = nbr_f.reshape(bs, p, _K)
    filled = fil_f.reshape(bs, p, _K) != 0
    ctr = jnp.broadcast_to(
        jnp.arange(p, dtype=jnp.int32)[None, :, None], (bs, p, _K))
    edges = jnp.stack([nbr, ctr], axis=-1)
    return edges, filled, xyz


# speedup vs baseline: 1.0035x; 1.0035x over previous
import functools

import jax
import jax.numpy as jnp
from jax import lax
from jax.experimental import pallas as pl
from jax.experimental.pallas import tpu as pltpu
from jax.experimental.pallas import tpu_sc as plsc

_BS = 4
_P = 4096
_K = 32
_R2 = 0.2 * 0.2
_L = 16
_NW = 32
_CPW = _BS * _P // _NW
_WPB = _P // _CPW
_NV = _P // _L
_CHUNK = 16
_SUB = 16
_NCH = _NV // _CHUNK
_SCRATCH = 3 * _CHUNK * _L + _L


def _radius_body(xyz_hbm, nbr_hbm, fil_hbm, x_ref, y_ref, z_ref,
                 sc_ref, nbr_v, fil_v, cnt_ref):
    wid = lax.axis_index("s") * 2 + lax.axis_index("c")
    b = wid // _WPB
    c0 = (wid % _WPB) * _CPW

    pltpu.sync_copy(xyz_hbm.at[pl.ds((b * 3 + 0) * _P, _P)],
                    x_ref.at[pl.ds(0, _P)])
    pltpu.sync_copy(xyz_hbm.at[pl.ds((b * 3 + 1) * _P, _P)],
                    y_ref.at[pl.ds(0, _P)])
    pltpu.sync_copy(xyz_hbm.at[pl.ds((b * 3 + 2) * _P, _P)],
                    z_ref.at[pl.ds(0, _P)])

    lanes = lax.iota(jnp.int32, 16)

    def per_center(i, _):
        c = c0 + i
        cxv = jnp.full((16,), x_ref[pl.ds(c, _L)][0], jnp.float32)
        cyv = jnp.full((16,), y_ref[pl.ds(c, _L)][0], jnp.float32)
        czv = jnp.full((16,), z_ref[pl.ds(c, _L)][0], jnp.float32)

        def scan_chunk(base, cnt):
            ms = []
            for u in range(_CHUNK):
                off = base + u * _L
                px = x_ref[pl.ds(off, _L)]
                py = y_ref[pl.ds(off, _L)]
                pz = z_ref[pl.ds(off, _L)]
                dx = px - cxv
                dy = py - cyv
                dz = pz - czv
                ms.append(dx * dx + dy * dy + dz * dz < _R2)
            for u in range(_CHUNK):
                off = base + u * _L
                plsc.store_compressed(sc_ref.at[pl.ds(cnt, _L)],
                                      off + lanes, mask=ms[u])
                cnt = cnt + plsc.all_reduce_population_count(ms[u])[0]
            return cnt

        cnt = scan_chunk(0, 0)
        cnt = scan_chunk(_CHUNK * _L, cnt)
        cnt = scan_chunk(2 * _CHUNK * _L, cnt)
        cnt_ref[0] = cnt

        def chunk_step(ch, _c):
            @pl.when(cnt_ref[0] < _K)
            def _do_chunk():
                cnt_ref[0] = scan_chunk(ch * (_CHUNK * _L), cnt_ref[0])

            return 0

        lax.fori_loop(3, 6, chunk_step, 0)

        @pl.when(cnt_ref[0] < _K)
        def _tail():
            lax.fori_loop(6, _NCH, chunk_step, 0)
        cnt = jnp.full((16,), cnt_ref[0], jnp.int32)

        v0 = sc_ref[pl.ds(0, _L)]
        v1 = sc_ref[pl.ds(_L, _L)]
        first = jnp.full((16,), v0[0], jnp.int32)
        f0 = lax.shift_right_logical(lanes - cnt, 31)
        f1 = lax.shift_right_logical((lanes + _L) - cnt, 31)
        o = i * _K
        nbr_v[pl.ds(o, _L)] = v0 * f0 + first * (1 - f0)
        nbr_v[pl.ds(o + _L, _L)] = v1 * f1 + first * (1 - f1)
        fil_v[pl.ds(o, _L)] = f0
        fil_v[pl.ds(o + _L, _L)] = f1
        return 0

    lax.fori_loop(0, _CPW, per_center, 0)

    pltpu.sync_copy(nbr_v, nbr_hbm.at[pl.ds(wid * _CPW * _K, _CPW * _K)])
    pltpu.sync_copy(fil_v, fil_hbm.at[pl.ds(wid * _CPW * _K, _CPW * _K)])


_radius_sc = functools.partial(
    pl.kernel,
    mesh=plsc.VectorSubcoreMesh(core_axis_name="c", subcore_axis_name="s"),
    out_type=[
        jax.ShapeDtypeStruct((_BS * _P * _K,), jnp.int32),
        jax.ShapeDtypeStruct((_BS * _P * _K,), jnp.int32),
    ],
    scratch_types=[
        pltpu.VMEM((_P + _L,), jnp.float32),
        pltpu.VMEM((_P + _L,), jnp.float32),
        pltpu.VMEM((_P + _L,), jnp.float32),
        pltpu.VMEM((_SCRATCH,), jnp.int32),
        pltpu.VMEM((_CPW * _K,), jnp.int32),
        pltpu.VMEM((_CPW * _K,), jnp.int32),
        pltpu.SMEM((1,), jnp.int32),
    ],
    compiler_params=pltpu.CompilerParams(needs_layout_passes=False),
)(_radius_body)


def kernel(xyz):
    bs, p = xyz.shape[:2]
    xyz_t = jnp.transpose(xyz, (0, 2, 1)).reshape(-1)
    nbr_f, fil_f = _radius_sc(xyz_t)
    nbr = nbr_f.reshape(bs, p, _K)
    filled = fil_f.reshape(bs, p, _K) != 0
    ctr = jnp.broadcast_to(
        jnp.arange(p, dtype=jnp.int32)[None, :, None], (bs, p, _K))
    edges = jnp.stack([nbr, ctr], axis=-1)
    return edges, filled, xyz
